# Initial kernel scaffold; baseline (speedup 1.0000x reference)
import jax
import jax.numpy as jnp
from jax.experimental import pallas as pl


def kernel(track_context, album_context, artist_context, next_track, next_album, next_artist, neg_track, neg_album, neg_artist, track_table, album_table, artist_table):
    raise NotImplementedError("write your pallas kernel here")


# speedup vs baseline: 1.1116x; 1.1116x over previous
import functools

import jax
import jax.numpy as jnp
from jax import lax
from jax.experimental import pallas as pl
from jax.experimental.pallas import tpu as pltpu
from jax.experimental.pallas import tpu_sc as plsc

_NCTX = 200
_NNEXT = 4096
_NNEG = 16384
_NROWS = _NCTX + _NNEXT + _NNEG
_FEAT = 64
_B = 20736
_NW = 32
_BPW = _B // _NW
_CHUNKS = [(o, min(128, _BPW - o)) for o in range(0, _BPW, 128)]

_mesh = plsc.VectorSubcoreMesh(core_axis_name="c", subcore_axis_name="s")


@functools.partial(
    pl.kernel,
    mesh=_mesh,
    out_type=[jax.ShapeDtypeStruct((_B, _FEAT), jnp.float32)] * 3,
    scratch_types=[
        pltpu.VMEM((_BPW,), jnp.int32),
        pltpu.VMEM((_BPW, _FEAT), jnp.float32),
        pltpu.SemaphoreType.DMA,
    ],
)
def _sc_gather(tab_t, tab_a, tab_r, idx_t, idx_a, idx_r,
               out_t, out_a, out_r, idx_v, rows_v, sem):
    wid = lax.axis_index("s") * 2 + lax.axis_index("c")
    base = wid * _BPW
    for tab, idx_hbm, out in ((tab_t, idx_t, out_t),
                              (tab_a, idx_a, out_a),
                              (tab_r, idx_r, out_r)):
        pltpu.sync_copy(idx_hbm.at[pl.ds(base, _BPW)], idx_v)
        cps = [pltpu.async_copy(tab.at[idx_v.at[pl.ds(o, n)]],
                                rows_v.at[pl.ds(o, n)], sem)
               for o, n in _CHUNKS]
        for cp in cps:
            cp.wait()
        pltpu.sync_copy(rows_v, out.at[pl.ds(base, _BPW)])


_RB = 1152
_CTXP = 256


def _tc_body(et, ea, er, ct, ca, cr, aff, nrm):
    dn = (((1,), (1,)), ((), ()))
    s = lax.dot_general(et[...], ct[...], dn)
    s += lax.dot_general(ea[...], ca[...], dn)
    s += lax.dot_general(er[...], cr[...], dn)
    col = lax.broadcasted_iota(jnp.int32, (_RB, _CTXP), 1)
    s = jnp.where(col < _NCTX, s, -jnp.inf)
    aff[...] = jnp.max(s, axis=1)[None, None, :]
    nrm[...] = jnp.sqrt(jnp.sum(et[...] * et[...], 1)
                        + jnp.sum(ea[...] * ea[...], 1)
                        + jnp.sum(er[...] * er[...], 1))[None, None, :]


_eb = pl.BlockSpec((_RB, _FEAT), lambda i: (i, 0))
_cb = pl.BlockSpec((_CTXP, _FEAT), lambda i: (0, 0))
_ob = pl.BlockSpec((1, 1, _RB), lambda i: (i, 0, 0))

_tc_compute = pl.pallas_call(
    _tc_body,
    grid=(_B // _RB,),
    in_specs=[_eb, _eb, _eb, _cb, _cb, _cb],
    out_specs=[_ob, _ob],
    out_shape=[jax.ShapeDtypeStruct((_B // _RB, 1, _RB), jnp.float32)] * 2,
)


def kernel(track_context, album_context, artist_context,
           next_track, next_album, next_artist,
           neg_track, neg_album, neg_artist,
           track_table, album_table, artist_table):
    pad = jnp.zeros((_B - _NROWS,), jnp.int32)
    idx_t = jnp.concatenate([track_context.astype(jnp.int32),
                             next_track.astype(jnp.int32),
                             neg_track.astype(jnp.int32), pad])
    idx_a = jnp.concatenate([album_context.astype(jnp.int32),
                             next_album.astype(jnp.int32),
                             neg_album.astype(jnp.int32), pad])
    idx_r = jnp.concatenate([artist_context.astype(jnp.int32),
                             next_artist.astype(jnp.int32),
                             neg_artist.astype(jnp.int32), pad])
    e_t = jnp.take(track_table, idx_t, axis=0)
    e_a = jnp.take(album_table, idx_a, axis=0)
    e_r = jnp.take(artist_table, idx_r, axis=0)
    aff, nrm = _tc_compute(e_t, e_a, e_r, e_t, e_a, e_r)
    aff = aff.reshape(_B)
    nrm = nrm.reshape(_B)
    return (aff[_NCTX:_NCTX + _NNEXT],
            aff[_NCTX + _NNEXT:_NROWS],
            nrm[:_NROWS])
